# baseline (speedup 1.0000x reference)
import functools

import jax
import jax.numpy as jnp
from jax import lax
from jax.experimental import pallas as pl
from jax.experimental.pallas import tpu as pltpu
from jax.experimental.pallas import tpu_sc as plsc

D = 64
HW = 256
K = 3
IDX_MINOR = 128
IDX_CHUNKS = (K * HW) // IDX_MINOR
NW = 32


def _sc_bow_embed(idx, table, batch):
    nb = batch // NW

    mesh = plsc.VectorSubcoreMesh(core_axis_name="c", subcore_axis_name="s")

    @functools.partial(
        pl.kernel,
        out_type=jax.ShapeDtypeStruct((batch, D, 16, 16), jnp.float32),
        mesh=mesh,
        compiler_params=pltpu.CompilerParams(
            needs_layout_passes=False, use_tc_tiling_on_sc=False),
        scratch_types=[
            pltpu.VMEM((2, IDX_CHUNKS, IDX_MINOR), jnp.int32),
            pltpu.VMEM((2, K * HW, D), jnp.float32),
            pltpu.VMEM((D, 16, 16), jnp.float32),
            pltpu.SemaphoreType.DMA,
            pltpu.SemaphoreType.DMA,
            pltpu.SemaphoreType.DMA,
        ],
    )
    def body(idx_hbm, table_hbm, out_hbm, idx_v, rows_v, out_t, g0, g1, osem):
        wid = lax.axis_index("s") * 2 + lax.axis_index("c")
        b0 = wid * nb
        lane = lax.iota(jnp.int32, 16)
        third = jnp.float32(1.0 / 3.0)
        e_cols = [c * 16 + lane for c in range(4)]
        gsem = (g0, g1)

        def fire(buf):
            for j in range(IDX_CHUNKS):
                pltpu.async_copy(
                    table_hbm.at[idx_v.at[buf, j]],
                    rows_v.at[buf, pl.ds(j * IDX_MINOR, IDX_MINOR)],
                    gsem[buf],
                )

        def drain(buf):
            for j in range(IDX_CHUNKS):
                pltpu.make_async_copy(
                    table_hbm.at[idx_v.at[buf, j]],
                    rows_v.at[buf, pl.ds(j * IDX_MINOR, IDX_MINOR)],
                    gsem[buf],
                ).wait()

        def out_start(i):
            pltpu.async_copy(out_t, out_hbm.at[b0 + i], osem)

        def out_wait(i):
            pltpu.make_async_copy(out_t, out_hbm.at[b0 + i], osem).wait()

        def compute(buf):
            rv = rows_v.at[buf]

            def t_body(t, _):
                h_idx = jnp.zeros((16,), jnp.int32) + t
                for d in range(16):
                    m = (lane + d) & 15
                    row_t = t * 16 + m
                    for c in range(4):
                        g = [plsc.load_gather(rv, [k * HW + row_t, e_cols[c]])
                             for k in range(K)]
                        v = (g[0] + g[1] + g[2]) * third
                        plsc.store_scatter(out_t, [e_cols[c], h_idx, m], v)
                return 0

            lax.fori_loop(0, 16, t_body, 0)

        pltpu.sync_copy(idx_hbm.at[b0], idx_v.at[0])
        fire(0)

        def pair_body(p, _):
            i = p * 2
            for par in (0, 1):
                ii = i + par
                nxt = ii + 1

                @pl.when(nxt < nb)
                def _():
                    pltpu.sync_copy(idx_hbm.at[b0 + nxt], idx_v.at[1 - par])
                    fire(1 - par)

                drain(par)

                @pl.when(ii > 0)
                def _():
                    out_wait(ii - 1)

                compute(par)
                out_start(ii)
            return 0

        lax.fori_loop(0, nb // 2, pair_body, 0)
        out_wait(nb - 1)

    return body(idx, table)


def kernel(inputs, table):
    b, k, h, w = inputs.shape
    idx = inputs.reshape(b, IDX_CHUNKS, IDX_MINOR)
    return _sc_bow_embed(idx, table, b)


# speedup vs baseline: 3.1265x; 3.1265x over previous
import functools

import jax
import jax.numpy as jnp
from jax import lax
from jax.experimental import pallas as pl
from jax.experimental.pallas import tpu as pltpu
from jax.experimental.pallas import tpu_sc as plsc

D = 64
HW = 256
K = 3
IDX_MINOR = 128
IDX_CHUNKS = (K * HW) // IDX_MINOR
OUT_PAD = 257
NW = 32


def _sc_bow_embed(idx, table, batch):
    nb = batch // NW

    mesh = plsc.VectorSubcoreMesh(core_axis_name="c", subcore_axis_name="s")

    @functools.partial(
        pl.kernel,
        out_type=jax.ShapeDtypeStruct((batch, D, HW), jnp.float32),
        mesh=mesh,
        compiler_params=pltpu.CompilerParams(
            needs_layout_passes=False, use_tc_tiling_on_sc=False),
        scratch_types=[
            pltpu.VMEM((2, IDX_CHUNKS, IDX_MINOR), jnp.int32),
            pltpu.VMEM((2, K * HW, D), jnp.float32),
            pltpu.VMEM((D, OUT_PAD), jnp.float32),
            pltpu.SemaphoreType.DMA,
            pltpu.SemaphoreType.DMA,
            pltpu.SemaphoreType.DMA,
        ],
    )
    def body(idx_hbm, table_hbm, out_hbm, idx_v, rows_v, out_t, g0, g1, osem):
        wid = lax.axis_index("s") * 2 + lax.axis_index("c")
        b0 = wid * nb
        lane = lax.iota(jnp.int32, 16)
        third = jnp.float32(1.0 / 3.0)
        e_rows = [c * 16 + lane for c in range(4)]
        gsem = (g0, g1)

        def fire(buf):
            for j in range(IDX_CHUNKS):
                pltpu.async_copy(
                    table_hbm.at[idx_v.at[buf, j]],
                    rows_v.at[buf, pl.ds(j * IDX_MINOR, IDX_MINOR)],
                    gsem[buf],
                )

        def drain(buf):
            for j in range(IDX_CHUNKS):
                pltpu.make_async_copy(
                    table_hbm.at[idx_v.at[buf, j]],
                    rows_v.at[buf, pl.ds(j * IDX_MINOR, IDX_MINOR)],
                    gsem[buf],
                ).wait()

        def out_start(i):
            pltpu.async_copy(
                out_t.at[:, pl.ds(0, HW)], out_hbm.at[b0 + i], osem)

        def out_wait(i):
            pltpu.make_async_copy(
                out_t.at[:, pl.ds(0, HW)], out_hbm.at[b0 + i], osem).wait()

        def compute(buf):
            rv = rows_v.at[buf]

            @plsc.parallel_loop(0, HW, 1, unroll=4)
            def _(hw):
                col = jnp.zeros((16,), jnp.int32) + hw
                for c in range(4):
                    sl = pl.ds(c * 16, 16)
                    v = (rv[hw, sl] + rv[HW + hw, sl]
                         + rv[2 * HW + hw, sl]) * third
                    plsc.store_scatter(out_t, [e_rows[c], col], v)

        pltpu.sync_copy(idx_hbm.at[b0], idx_v.at[0])
        fire(0)

        def pair_body(p, _):
            i = p * 2
            for par in (0, 1):
                ii = i + par
                nxt = ii + 1

                @pl.when(nxt < nb)
                def _():
                    pltpu.sync_copy(idx_hbm.at[b0 + nxt], idx_v.at[1 - par])
                    fire(1 - par)

                drain(par)

                @pl.when(ii > 0)
                def _():
                    out_wait(ii - 1)

                compute(par)
                out_start(ii)
            return 0

        lax.fori_loop(0, nb // 2, pair_body, 0)
        out_wait(nb - 1)

    return body(idx, table)


def kernel(inputs, table):
    b, k, h, w = inputs.shape
    idx = inputs.reshape(b, IDX_CHUNKS, IDX_MINOR)
    out = _sc_bow_embed(idx, table, b)
    return out.reshape(b, D, h, w)


# speedup vs baseline: 3.7702x; 1.2059x over previous
import functools

import jax
import jax.numpy as jnp
from jax import lax
from jax.experimental import pallas as pl
from jax.experimental.pallas import tpu as pltpu
from jax.experimental.pallas import tpu_sc as plsc

D = 64
HW = 256
K = 3
BB = 256
IDX_MINOR = 128
OUT_PAD = 257
NW = 32


def _sc_bow_embed(idx, table, batch):
    n_units = HW * (batch // BB) // NW
    chunks = batch // BB

    mesh = plsc.VectorSubcoreMesh(core_axis_name="c", subcore_axis_name="s")

    @functools.partial(
        pl.kernel,
        out_type=jax.ShapeDtypeStruct((D, 16, 16, batch), jnp.float32),
        mesh=mesh,
        compiler_params=pltpu.CompilerParams(
            needs_layout_passes=False, use_tc_tiling_on_sc=False),
        scratch_types=[
            pltpu.VMEM((2, K, 2, IDX_MINOR), jnp.int32),
            pltpu.VMEM((2, K * BB, D), jnp.float32),
            pltpu.VMEM((D, OUT_PAD), jnp.float32),
            pltpu.SemaphoreType.DMA,
            pltpu.SemaphoreType.DMA,
            pltpu.SemaphoreType.DMA,
        ],
    )
    def body(idx_hbm, table_hbm, out_hbm, idx_v, rows_v, out_t, g0, g1, osem):
        wid = lax.axis_index("s") * 2 + lax.axis_index("c")
        lane = lax.iota(jnp.int32, 16)
        third = jnp.float32(1.0 / 3.0)
        e_rows = [c * 16 + lane for c in range(4)]
        gsem = (g0, g1)

        def unit_hwb(u):
            hw = wid * (n_units // chunks) + (u // chunks)
            cb = u % chunks
            return hw // 16, hw % 16, cb

        def idx_copy(u, buf):
            h, w, cb = unit_hwb(u)
            pltpu.sync_copy(
                idx_hbm.at[:, h, w, pl.ds(cb * 2, 2)], idx_v.at[buf])

        def fire(buf):
            for k in range(K):
                for j in range(2):
                    pltpu.async_copy(
                        table_hbm.at[idx_v.at[buf, k, j]],
                        rows_v.at[buf, pl.ds(k * BB + j * IDX_MINOR,
                                             IDX_MINOR)],
                        gsem[buf],
                    )

        def drain(buf):
            for k in range(K):
                for j in range(2):
                    pltpu.make_async_copy(
                        table_hbm.at[idx_v.at[buf, k, j]],
                        rows_v.at[buf, pl.ds(k * BB + j * IDX_MINOR,
                                             IDX_MINOR)],
                        gsem[buf],
                    ).wait()

        def out_dma(u):
            h, w, cb = unit_hwb(u)
            return pltpu.make_async_copy(
                out_t.at[:, pl.ds(0, BB)],
                out_hbm.at[:, h, w, pl.ds(cb * BB, BB)],
                osem,
            )

        def compute(buf):
            rv = rows_v.at[buf]

            @plsc.parallel_loop(0, BB, 1, unroll=4)
            def _(p):
                col = jnp.zeros((16,), jnp.int32) + p
                for c in range(4):
                    sl = pl.ds(c * 16, 16)
                    v = (rv[p, sl] + rv[BB + p, sl]
                         + rv[2 * BB + p, sl]) * third
                    plsc.store_scatter(out_t, [e_rows[c], col], v)

        idx_copy(0, 0)
        fire(0)

        def pair_body(pr, _):
            i = pr * 2
            for par in (0, 1):
                u = i + par
                nxt = u + 1

                @pl.when(nxt < n_units)
                def _():
                    idx_copy(nxt, 1 - par)
                    fire(1 - par)

                drain(par)

                @pl.when(u > 0)
                def _():
                    out_dma(u - 1).wait()

                compute(par)
                out_dma(u).start()
            return 0

        lax.fori_loop(0, n_units // 2, pair_body, 0)
        out_dma(n_units - 1).wait()

    return body(idx, table)


def kernel(inputs, table):
    b, k, h, w = inputs.shape
    idx = inputs.transpose(1, 2, 3, 0).reshape(k, h, w, b // 128, 128)
    out = _sc_bow_embed(idx, table, b)
    return out.transpose(3, 0, 1, 2)


# speedup vs baseline: 5.0917x; 1.3505x over previous
import functools

import jax
import jax.numpy as jnp
from jax import lax
from jax.experimental import pallas as pl
from jax.experimental.pallas import tpu as pltpu
from jax.experimental.pallas import tpu_sc as plsc

D = 64
HW = 256
K = 3
BB = 256
IDX_MINOR = 128
OUT_PAD = 257
NW = 32


def _sc_bow_embed(idx, table, batch):
    n_units = HW * (batch // BB) // NW
    chunks = batch // BB

    mesh = plsc.VectorSubcoreMesh(core_axis_name="c", subcore_axis_name="s")

    @functools.partial(
        pl.kernel,
        out_type=jax.ShapeDtypeStruct((D, 16, 2, batch // 128, 8, 128),
                                      jnp.float32),
        mesh=mesh,
        compiler_params=pltpu.CompilerParams(
            needs_layout_passes=False, use_tc_tiling_on_sc=False),
        scratch_types=[
            pltpu.VMEM((2, K, 2, IDX_MINOR), jnp.int32),
            pltpu.VMEM((2, K * BB, D), jnp.float32),
            pltpu.VMEM((D, OUT_PAD), jnp.float32),
            pltpu.SemaphoreType.DMA,
            pltpu.SemaphoreType.DMA,
            pltpu.SemaphoreType.DMA,
        ],
    )
    def body(idx_hbm, table_hbm, out_hbm, idx_v, rows_v, out_t, g0, g1, osem):
        wid = lax.axis_index("s") * 2 + lax.axis_index("c")
        lane = lax.iota(jnp.int32, 16)
        third = jnp.float32(1.0 / 3.0)
        e_rows = [c * 16 + lane for c in range(4)]
        gsem = (g0, g1)

        def unit_hwb(u):
            hw = wid * (n_units // chunks) + (u // chunks)
            cb = u % chunks
            return hw // 16, hw % 16, cb

        def idx_copy(u, buf):
            h, w, cb = unit_hwb(u)
            pltpu.sync_copy(
                idx_hbm.at[:, h, w, pl.ds(cb * 2, 2)], idx_v.at[buf])

        def fire(buf):
            for k in range(K):
                for j in range(2):
                    pltpu.async_copy(
                        table_hbm.at[idx_v.at[buf, k, j]],
                        rows_v.at[buf, pl.ds(k * BB + j * IDX_MINOR,
                                             IDX_MINOR)],
                        gsem[buf],
                    )

        def drain(buf):
            for k in range(K):
                for j in range(2):
                    pltpu.make_async_copy(
                        table_hbm.at[idx_v.at[buf, k, j]],
                        rows_v.at[buf, pl.ds(k * BB + j * IDX_MINOR,
                                             IDX_MINOR)],
                        gsem[buf],
                    ).wait()

        def out_dmas(u):
            h, w, cb = unit_hwb(u)
            wt, w8 = w // 8, w % 8
            return [
                pltpu.make_async_copy(
                    out_t.at[:, pl.ds(j * IDX_MINOR, IDX_MINOR)],
                    out_hbm.at[:, h, wt, 2 * cb + j, w8],
                    osem,
                )
                for j in range(2)
            ]

        def compute(buf):
            rv = rows_v.at[buf]

            @plsc.parallel_loop(0, BB, 1, unroll=4)
            def _(p):
                col = jnp.zeros((16,), jnp.int32) + p
                for c in range(4):
                    sl = pl.ds(c * 16, 16)
                    v = (rv[p, sl] + rv[BB + p, sl]
                         + rv[2 * BB + p, sl]) * third
                    plsc.store_scatter(out_t, [e_rows[c], col], v)

        idx_copy(0, 0)
        fire(0)

        def pair_body(pr, _):
            i = pr * 2
            for par in (0, 1):
                u = i + par
                nxt = u + 1

                @pl.when(nxt < n_units)
                def _():
                    idx_copy(nxt, 1 - par)
                    fire(1 - par)

                drain(par)

                @pl.when(u > 0)
                def _():
                    for cp in out_dmas(u - 1):
                        cp.wait()

                compute(par)
                for cp in out_dmas(u):
                    cp.start()
            return 0

        lax.fori_loop(0, n_units // 2, pair_body, 0)
        for cp in out_dmas(n_units - 1):
            cp.wait()

    return body(idx, table)


def kernel(inputs, table):
    b, k, h, w = inputs.shape
    idx = inputs.transpose(1, 2, 3, 0).reshape(k, h, w, b // 128, 128)
    out = _sc_bow_embed(idx, table, b)
    out = out.transpose(3, 5, 0, 1, 2, 4)
    return out.reshape(b, D, h, w)
